# baseline (speedup 1.0000x reference)
import jax
import jax.numpy as jnp
from jax import lax
from jax.experimental import pallas as pl
from jax.experimental.pallas import tpu as pltpu
from jax.experimental.pallas import tpu_sc as plsc

NC = 2
NS = 16
NW = NC * NS

EMBED = 128
CHUNK = 128
NBUF = 5


def _embed_body(tok_hbm, table_hbm, out_hbm, idx_v, bufs, gsem, wsem,
                n_chunk):
    wid = lax.axis_index("s") * NC + lax.axis_index("c")
    pltpu.sync_copy(tok_hbm.at[wid], idx_v)
    row0 = wid * (n_chunk * CHUNK)

    def g_copy(j, b):
        return pltpu.make_async_copy(
            table_hbm.at[idx_v.at[j]], bufs.at[b], gsem.at[b])

    def w_copy(j, b):
        return pltpu.make_async_copy(
            bufs.at[b], out_hbm.at[pl.ds(row0 + j * CHUNK, CHUNK)],
            wsem.at[b])

    for b in range(NBUF):
        g_copy(b, b).start()

    @pl.loop(0, n_chunk, step=NBUF)
    def _(j0):
        for b in range(NBUF):
            j = j0 + b
            g_copy(j, b).wait()
            w_copy(j, b).start()

            @pl.when(j + NBUF < n_chunk)
            def _():
                w_copy(j, b).wait()
                g_copy(j + NBUF, b).start()

    for b in range(NBUF):
        w_copy(n_chunk - NBUF + b, b).wait()


def kernel(tokens, table):
    n_seq, seq_len = tokens.shape
    n_tok = n_seq * seq_len
    assert n_tok % (NW * CHUNK) == 0
    n_chunk = n_tok // (NW * CHUNK)
    assert n_chunk % NBUF == 0
    tok_cube = tokens.astype(jnp.int32).reshape(NW, n_chunk, CHUNK)

    mesh = plsc.VectorSubcoreMesh(
        core_axis_name="c", subcore_axis_name="s",
        num_cores=NC, num_subcores=NS)

    def body(tok_hbm, table_hbm, out_hbm, idx_v, bufs, gsem, wsem):
        _embed_body(tok_hbm, table_hbm, out_hbm, idx_v, bufs, gsem, wsem,
                    n_chunk)

    out = pl.kernel(
        body,
        out_type=jax.ShapeDtypeStruct((n_tok, EMBED), jnp.float32),
        mesh=mesh,
        compiler_params=pltpu.CompilerParams(use_tc_tiling_on_sc=True),
        scratch_types=[
            pltpu.VMEM((n_chunk, CHUNK), jnp.int32),
            pltpu.VMEM((NBUF, CHUNK, EMBED), jnp.float32),
            pltpu.SemaphoreType.DMA((NBUF,)),
            pltpu.SemaphoreType.DMA((NBUF,)),
        ],
    )(tok_cube, table)
    return out.reshape(n_seq, seq_len, EMBED)


# speedup vs baseline: 1.7857x; 1.7857x over previous
import jax
import jax.numpy as jnp
from jax import lax
from jax.experimental import pallas as pl
from jax.experimental.pallas import tpu as pltpu
from jax.experimental.pallas import tpu_sc as plsc

NC = 2
NS = 16
NW = NC * NS

EMBED = 128
PAIR = 2
NBUF = 4


def _embed_body(tok_hbm, table_hbm, out_hbm, idx_v, bufs, gsem, wsem,
                n_pair, seq_len):
    wid = lax.axis_index("s") * NC + lax.axis_index("c")
    pltpu.sync_copy(tok_hbm.at[wid], idx_v)
    seq0 = wid * n_pair * PAIR

    def g_copy(j, b):
        return pltpu.make_async_copy(
            table_hbm.at[idx_v.at[j, pl.ds(0, PAIR * seq_len)]],
            bufs.at[b], gsem.at[b])

    def w_copy(j, b, h):
        return pltpu.make_async_copy(
            bufs.at[b, pl.ds(h * seq_len, seq_len)],
            out_hbm.at[seq0 + PAIR * j + h], wsem.at[b, h])

    for b in range(NBUF):
        g_copy(b, b).start()

    @pl.loop(0, n_pair, step=NBUF)
    def _(j0):
        for b in range(NBUF):
            j = j0 + b
            g_copy(j, b).wait()
            for h in range(PAIR):
                w_copy(j, b, h).start()

            @pl.when(j + NBUF < n_pair)
            def _():
                for h in range(PAIR):
                    w_copy(j, b, h).wait()
                g_copy(j + NBUF, b).start()

    for b in range(NBUF):
        for h in range(PAIR):
            w_copy(n_pair - NBUF + b, b, h).wait()


def kernel(tokens, table):
    n_seq, seq_len = tokens.shape
    assert PAIR * seq_len <= 128
    assert n_seq % (NW * PAIR) == 0
    n_pair = n_seq // (NW * PAIR)
    assert n_pair % NBUF == 0
    tok_rows = tokens.astype(jnp.int32).reshape(NW * n_pair, PAIR * seq_len)
    tok_pad = jnp.zeros((NW * n_pair, 128), jnp.int32)
    tok_pad = lax.dynamic_update_slice(tok_pad, tok_rows, (0, 0))
    tok_cube = tok_pad.reshape(NW, n_pair, 128)

    mesh = plsc.VectorSubcoreMesh(
        core_axis_name="c", subcore_axis_name="s",
        num_cores=NC, num_subcores=NS)

    def body(tok_hbm, table_hbm, out_hbm, idx_v, bufs, gsem, wsem):
        _embed_body(tok_hbm, table_hbm, out_hbm, idx_v, bufs, gsem, wsem,
                    n_pair, seq_len)

    out = pl.kernel(
        body,
        out_type=jax.ShapeDtypeStruct((n_seq, seq_len, EMBED), jnp.float32),
        mesh=mesh,
        compiler_params=pltpu.CompilerParams(use_tc_tiling_on_sc=True),
        scratch_types=[
            pltpu.VMEM((n_pair, 128), jnp.int32),
            pltpu.VMEM((NBUF, PAIR * seq_len, EMBED), jnp.float32),
            pltpu.SemaphoreType.DMA((NBUF,)),
            pltpu.SemaphoreType.DMA((NBUF, PAIR)),
        ],
    )(tok_cube, table)
    return out


# speedup vs baseline: 1.7886x; 1.0016x over previous
import jax
import jax.numpy as jnp
from jax import lax
from jax.experimental import pallas as pl
from jax.experimental.pallas import tpu as pltpu
from jax.experimental.pallas import tpu_sc as plsc

NC = 2
NS = 16
NW = NC * NS

EMBED = 128
NBUF = 8


def _embed_body(tok_hbm, table_hbm, out_hbm, idx_v, bufs, gsem, wsem,
                seq_per_w, seq_len):
    wid = lax.axis_index("s") * NC + lax.axis_index("c")
    pltpu.sync_copy(tok_hbm.at[wid], idx_v)
    seq0 = wid * seq_per_w

    def g_copy(j, b):
        return pltpu.make_async_copy(
            table_hbm.at[idx_v.at[j, pl.ds(0, seq_len)]], bufs.at[b],
            gsem.at[b])

    def w_copy(j, b):
        return pltpu.make_async_copy(
            bufs.at[b], out_hbm.at[seq0 + j], wsem.at[b])

    for b in range(NBUF):
        g_copy(b, b).start()

    @pl.loop(0, seq_per_w, step=NBUF)
    def _(j0):
        for b in range(NBUF):
            j = j0 + b
            g_copy(j, b).wait()
            w_copy(j, b).start()

            @pl.when(j + NBUF < seq_per_w)
            def _():
                w_copy(j, b).wait()
                g_copy(j + NBUF, b).start()

    for b in range(NBUF):
        w_copy(seq_per_w - NBUF + b, b).wait()


def kernel(tokens, table):
    n_seq, seq_len = tokens.shape
    assert n_seq % NW == 0
    seq_per_w = n_seq // NW
    assert seq_per_w % NBUF == 0
    tok_pad = jnp.zeros((n_seq, 128), jnp.int32)
    tok_pad = lax.dynamic_update_slice(
        tok_pad, tokens.astype(jnp.int32), (0, 0))
    tok_cube = tok_pad.reshape(NW, seq_per_w, 128)

    mesh = plsc.VectorSubcoreMesh(
        core_axis_name="c", subcore_axis_name="s",
        num_cores=NC, num_subcores=NS)

    def body(tok_hbm, table_hbm, out_hbm, idx_v, bufs, gsem, wsem):
        _embed_body(tok_hbm, table_hbm, out_hbm, idx_v, bufs, gsem, wsem,
                    seq_per_w, seq_len)

    out = pl.kernel(
        body,
        out_type=jax.ShapeDtypeStruct((n_seq, seq_len, EMBED), jnp.float32),
        mesh=mesh,
        compiler_params=pltpu.CompilerParams(use_tc_tiling_on_sc=True),
        scratch_types=[
            pltpu.VMEM((seq_per_w, 128), jnp.int32),
            pltpu.VMEM((NBUF, seq_len, EMBED), jnp.float32),
            pltpu.SemaphoreType.DMA((NBUF,)),
            pltpu.SemaphoreType.DMA((NBUF,)),
        ],
    )(tok_cube, table)
    return out
